# baseline (speedup 1.0000x reference)
import functools

import jax
import jax.numpy as jnp
from jax import lax
from jax.experimental import pallas as pl
from jax.experimental.pallas import tpu as pltpu
from jax.experimental.pallas import tpu_sc as plsc

_NUM_WORKERS = 32
_LANES = 16
_E_BLK = 80


def _pack_pairs(x_bf16):
    d = x_bf16.shape[1]
    lo = lax.bitcast_convert_type(x_bf16[:, :d // 2], jnp.uint16)
    hi = lax.bitcast_convert_type(x_bf16[:, d // 2:], jnp.uint16)
    packed = lo.astype(jnp.uint32) | (hi.astype(jnp.uint32) << 16)
    return packed.astype(jnp.int32)


def _prep_body(x_ref, r_ref, zo_ref, ro_ref):
    x = x_ref[...]
    n = jnp.sqrt(jnp.sum(x * x, axis=1, keepdims=True))
    z = (x / jnp.maximum(n, 1e-12)).astype(jnp.bfloat16)
    zo_ref[...] = _pack_pairs(z)
    ro_ref[...] = _pack_pairs(r_ref[...].astype(jnp.bfloat16))


def _prep_tables_tc(node_emb, rel_emb):
    nn, d = node_emb.shape
    nr = rel_emb.shape[0]
    return pl.pallas_call(
        _prep_body,
        out_shape=(jax.ShapeDtypeStruct((nn, d // 2), jnp.int32),
                   jax.ShapeDtypeStruct((nr, d // 2), jnp.int32)),
    )(node_emb, rel_emb)


@functools.cache
def _make_sc_score(num_edges, hidden):
    epw = num_edges // _NUM_WORKERS
    nblk = epw // _E_BLK
    hw = hidden // 2
    nch = hw // _LANES
    mesh = plsc.VectorSubcoreMesh(core_axis_name="c", subcore_axis_name="s")

    @functools.partial(
        pl.kernel,
        out_type=jax.ShapeDtypeStruct((num_edges,), jnp.float32),
        mesh=mesh,
        compiler_params=pltpu.CompilerParams(needs_layout_passes=False,
                                             use_tc_tiling_on_sc=False),
        scratch_types=[
            pltpu.VMEM((epw,), jnp.int32),
            pltpu.VMEM((epw,), jnp.int32),
            pltpu.VMEM((epw,), jnp.int32),
            pltpu.VMEM((_E_BLK, hw), jnp.int32),
            pltpu.VMEM((_E_BLK, hw), jnp.int32),
            pltpu.VMEM((_E_BLK, hw), jnp.int32),
            pltpu.VMEM((_E_BLK, hw), jnp.int32),
            pltpu.VMEM((_E_BLK, hw), jnp.int32),
            pltpu.VMEM((_E_BLK, hw), jnp.int32),
            pltpu.VMEM((epw,), jnp.float32),
            pltpu.VMEM((_LANES * (_LANES + 1),), jnp.float32),
            pltpu.SemaphoreType.DMA,
            pltpu.SemaphoreType.DMA,
        ],
    )
    def sc_score(z_hbm, rel_hbm, ei_hbm, typ_hbm, out_hbm,
                 si_v, di_v, ri_v, sr_a, dr_a, rr_a, sr_b, dr_b, rr_b,
                 out_v, acc_v, sem_a, sem_b):
        wid = lax.axis_index("s") * 2 + lax.axis_index("c")
        base = wid * epw

        pltpu.sync_copy(ei_hbm.at[0, pl.ds(base, epw)], si_v)
        pltpu.sync_copy(ei_hbm.at[1, pl.ds(base, epw)], di_v)
        pltpu.sync_copy(typ_hbm.at[pl.ds(base, epw)], ri_v)

        def issue(off, n, sr, dr, rr, sem):
            pltpu.async_copy(z_hbm.at[si_v.at[pl.ds(off, n)]],
                             sr.at[pl.ds(0, n)], sem)
            pltpu.async_copy(z_hbm.at[di_v.at[pl.ds(off, n)]],
                             dr.at[pl.ds(0, n)], sem)
            pltpu.async_copy(rel_hbm.at[ri_v.at[pl.ds(off, n)]],
                             rr.at[pl.ds(0, n)], sem)

        def wait(n, sr, dr, rr, sem):
            pltpu.make_async_copy(z_hbm.at[pl.ds(0, n)],
                                  sr.at[pl.ds(0, n)], sem).wait()
            pltpu.make_async_copy(z_hbm.at[pl.ds(0, n)],
                                  dr.at[pl.ds(0, n)], sem).wait()
            pltpu.make_async_copy(rel_hbm.at[pl.ds(0, n)],
                                  rr.at[pl.ds(0, n)], sem).wait()

        cols = lax.iota(jnp.int32, _LANES) * (_LANES + 1)

        def compute(off, n, sr, dr, rr):
            for g in range(n // _LANES):
                def ebody(e, carry):
                    re = g * _LANES + e
                    acc = None
                    for j in range(nch):
                        sv = plsc.bitcast(sr[re, pl.ds(j * _LANES, _LANES)],
                                          jnp.bfloat16)
                        dv = plsc.bitcast(dr[re, pl.ds(j * _LANES, _LANES)],
                                          jnp.bfloat16)
                        rv = plsc.bitcast(rr[re, pl.ds(j * _LANES, _LANES)],
                                          jnp.bfloat16)
                        t = sv * rv * dv
                        lo, hi = plsc.unpack(
                            t, format=plsc.PackFormat.INTERLEAVED,
                            preferred_element_type=jnp.float32)
                        u = lo + hi
                        acc = u if acc is None else acc + u
                    acc_v[pl.ds(e * (_LANES + 1), _LANES)] = acc
                    return carry

                lax.fori_loop(0, _LANES, ebody, 0, unroll=4)

                s0 = plsc.load_gather(acc_v, [cols])
                s1 = plsc.load_gather(acc_v, [cols + 1])
                for c in range(2, _LANES, 2):
                    s0 = s0 + plsc.load_gather(acc_v, [cols + c])
                    s1 = s1 + plsc.load_gather(acc_v, [cols + c + 1])
                out_v[pl.ds(off + g * _LANES, _LANES)] = s0 + s1

        issue(0, _E_BLK, sr_a, dr_a, rr_a, sem_a)

        def step(it, carry):
            i = 2 * it
            issue((i + 1) * _E_BLK, _E_BLK, sr_b, dr_b, rr_b, sem_b)
            wait(_E_BLK, sr_a, dr_a, rr_a, sem_a)
            compute(i * _E_BLK, _E_BLK, sr_a, dr_a, rr_a)
            issue((i + 2) * _E_BLK, _E_BLK, sr_a, dr_a, rr_a, sem_a)
            wait(_E_BLK, sr_b, dr_b, rr_b, sem_b)
            compute((i + 1) * _E_BLK, _E_BLK, sr_b, dr_b, rr_b)
            return carry

        lax.fori_loop(0, (nblk - 1) // 2, step, 0)
        wait(_E_BLK, sr_a, dr_a, rr_a, sem_a)
        compute((nblk - 1) * _E_BLK, _E_BLK, sr_a, dr_a, rr_a)

        pltpu.sync_copy(out_v, out_hbm.at[pl.ds(base, epw)])

    return sc_score


def kernel(node_emb, rel_emb, edge_index, edge_type):
    z_packed, rel_packed = _prep_tables_tc(node_emb, rel_emb)
    num_edges = edge_index.shape[1]
    sc_score = _make_sc_score(num_edges, node_emb.shape[1])
    return sc_score(z_packed, rel_packed, edge_index, edge_type)


# speedup vs baseline: 1.3782x; 1.3782x over previous
import functools

import jax
import jax.numpy as jnp
from jax import lax
from jax.experimental import pallas as pl
from jax.experimental.pallas import tpu as pltpu
from jax.experimental.pallas import tpu_sc as plsc

_NUM_WORKERS = 32
_LANES = 16
_E_BLK = 80


def _pack_pairs(x_bf16):
    d = x_bf16.shape[1]
    lo = lax.bitcast_convert_type(x_bf16[:, :d // 2], jnp.uint16)
    hi = lax.bitcast_convert_type(x_bf16[:, d // 2:], jnp.uint16)
    packed = lo.astype(jnp.uint32) | (hi.astype(jnp.uint32) << 16)
    return packed.astype(jnp.int32)


def _prep_body(x_ref, r_ref, zo_ref, ro_ref):
    x = x_ref[...]
    n = jnp.sqrt(jnp.sum(x * x, axis=1, keepdims=True))
    z = (x / jnp.maximum(n, 1e-12)).astype(jnp.bfloat16)
    zo_ref[...] = _pack_pairs(z)
    ro_ref[...] = _pack_pairs(r_ref[...].astype(jnp.bfloat16))


def _prep_tables_tc(node_emb, rel_emb):
    nn, d = node_emb.shape
    nr = rel_emb.shape[0]
    return pl.pallas_call(
        _prep_body,
        out_shape=(jax.ShapeDtypeStruct((nn, d // 2), jnp.int32),
                   jax.ShapeDtypeStruct((nr, d // 2), jnp.int32)),
    )(node_emb, rel_emb)


@functools.cache
def _make_sc_score(num_edges, hidden):
    epw = num_edges // _NUM_WORKERS
    nblk = epw // _E_BLK
    hw = hidden // 2
    nch = hw // _LANES
    mesh = plsc.VectorSubcoreMesh(core_axis_name="c", subcore_axis_name="s")

    @functools.partial(
        pl.kernel,
        out_type=jax.ShapeDtypeStruct((num_edges,), jnp.float32),
        mesh=mesh,
        compiler_params=pltpu.CompilerParams(needs_layout_passes=False,
                                             use_tc_tiling_on_sc=False),
        scratch_types=[
            pltpu.VMEM((epw,), jnp.int32),
            pltpu.VMEM((epw,), jnp.int32),
            pltpu.VMEM((epw,), jnp.int32),
            pltpu.VMEM((_E_BLK, hw), jnp.int32),
            pltpu.VMEM((_E_BLK, hw), jnp.int32),
            pltpu.VMEM((_E_BLK, hw), jnp.int32),
            pltpu.VMEM((_E_BLK, hw), jnp.int32),
            pltpu.VMEM((_E_BLK, hw), jnp.int32),
            pltpu.VMEM((_E_BLK, hw), jnp.int32),
            pltpu.VMEM((epw,), jnp.float32),
            pltpu.VMEM((_LANES * (_LANES + 1),), jnp.float32),
            pltpu.SemaphoreType.DMA,
            pltpu.SemaphoreType.DMA,
        ],
    )
    def sc_score(z_hbm, rel_hbm, ei_hbm, typ_hbm, out_hbm,
                 si_v, di_v, ri_v, sr_a, dr_a, rr_a, sr_b, dr_b, rr_b,
                 out_v, acc_v, sem_a, sem_b):
        wid = lax.axis_index("s") * 2 + lax.axis_index("c")
        base = wid * epw

        pltpu.sync_copy(ei_hbm.at[0, pl.ds(base, epw)], si_v)
        pltpu.sync_copy(ei_hbm.at[1, pl.ds(base, epw)], di_v)
        pltpu.sync_copy(typ_hbm.at[pl.ds(base, epw)], ri_v)

        def issue(off, n, sr, dr, rr, sem):
            pltpu.async_copy(z_hbm.at[si_v.at[pl.ds(off, n)]],
                             sr.at[pl.ds(0, n)], sem)
            pltpu.async_copy(z_hbm.at[di_v.at[pl.ds(off, n)]],
                             dr.at[pl.ds(0, n)], sem)
            pltpu.async_copy(rel_hbm.at[ri_v.at[pl.ds(off, n)]],
                             rr.at[pl.ds(0, n)], sem)

        def wait(n, sr, dr, rr, sem):
            pltpu.make_async_copy(z_hbm.at[pl.ds(0, n)],
                                  sr.at[pl.ds(0, n)], sem).wait()
            pltpu.make_async_copy(z_hbm.at[pl.ds(0, n)],
                                  dr.at[pl.ds(0, n)], sem).wait()
            pltpu.make_async_copy(rel_hbm.at[pl.ds(0, n)],
                                  rr.at[pl.ds(0, n)], sem).wait()

        cols = lax.iota(jnp.int32, _LANES) * (_LANES + 1)

        def compute(off, n, sr, dr, rr):
            for g in range(n // _LANES):
                def ebody(e, carry):
                    re = g * _LANES + e
                    acc = None
                    for j in range(nch):
                        sv = plsc.bitcast(sr[re, pl.ds(j * _LANES, _LANES)],
                                          jnp.bfloat16)
                        dv = plsc.bitcast(dr[re, pl.ds(j * _LANES, _LANES)],
                                          jnp.bfloat16)
                        rv = plsc.bitcast(rr[re, pl.ds(j * _LANES, _LANES)],
                                          jnp.bfloat16)
                        t = sv * rv * dv
                        lo, hi = plsc.unpack(
                            t, format=plsc.PackFormat.INTERLEAVED,
                            preferred_element_type=jnp.float32)
                        u = lo + hi
                        acc = u if acc is None else acc + u
                    acc_v[pl.ds(e * (_LANES + 1), _LANES)] = acc
                    return carry

                lax.fori_loop(0, _LANES, ebody, 0, unroll=2)

                s0 = plsc.load_gather(acc_v, [cols])
                s1 = plsc.load_gather(acc_v, [cols + 1])
                for c in range(2, _LANES, 2):
                    s0 = s0 + plsc.load_gather(acc_v, [cols + c])
                    s1 = s1 + plsc.load_gather(acc_v, [cols + c + 1])
                out_v[pl.ds(off + g * _LANES, _LANES)] = s0 + s1

        issue(0, _E_BLK, sr_a, dr_a, rr_a, sem_a)

        def step(it, carry):
            i = 2 * it
            issue((i + 1) * _E_BLK, _E_BLK, sr_b, dr_b, rr_b, sem_b)
            wait(_E_BLK, sr_a, dr_a, rr_a, sem_a)
            compute(i * _E_BLK, _E_BLK, sr_a, dr_a, rr_a)
            issue((i + 2) * _E_BLK, _E_BLK, sr_a, dr_a, rr_a, sem_a)
            wait(_E_BLK, sr_b, dr_b, rr_b, sem_b)
            compute((i + 1) * _E_BLK, _E_BLK, sr_b, dr_b, rr_b)
            return carry

        lax.fori_loop(0, (nblk - 1) // 2, step, 0)
        wait(_E_BLK, sr_a, dr_a, rr_a, sem_a)
        compute((nblk - 1) * _E_BLK, _E_BLK, sr_a, dr_a, rr_a)

        pltpu.sync_copy(out_v, out_hbm.at[pl.ds(base, epw)])

    return sc_score


def kernel(node_emb, rel_emb, edge_index, edge_type):
    z_packed, rel_packed = _prep_tables_tc(node_emb, rel_emb)
    num_edges = edge_index.shape[1]
    sc_score = _make_sc_score(num_edges, node_emb.shape[1])
    return sc_score(z_packed, rel_packed, edge_index, edge_type)
